# Initial kernel scaffold; baseline (speedup 1.0000x reference)
import jax
import jax.numpy as jnp
from jax.experimental import pallas as pl


def kernel(x, edge_index, params):
    raise NotImplementedError("write your pallas kernel here")


# speedup vs baseline: 2.5996x; 2.5996x over previous
import functools

import jax
import jax.numpy as jnp
from jax import lax
from jax.experimental import pallas as pl
from jax.experimental.pallas import tpu as pltpu
from jax.experimental.pallas import tpu_sc as plsc

_N = 10000
_DEG = 16
_E = _N * _DEG
_NEG = 0.01

_NW = 32
_CH = 1000
_TN = 1000


def _sc_gather(table, idx, D):
    E = idx.shape[0]
    per_w = E // _NW
    n_ch = per_w // _CH
    mesh = plsc.VectorSubcoreMesh(core_axis_name="c", subcore_axis_name="s")

    @functools.partial(
        pl.kernel,
        out_type=jax.ShapeDtypeStruct((E, D), jnp.float32),
        mesh=mesh,
        scratch_types=[
            pltpu.VMEM((_CH,), jnp.int32),
            pltpu.VMEM((_CH, D), jnp.float32),
            pltpu.SemaphoreType.DMA,
        ],
        compiler_params=pltpu.CompilerParams(use_tc_tiling_on_sc=False),
    )
    def gather_kernel(table_hbm, idx_hbm, out_hbm, idx_v, rows_v, sem):
        wid = lax.axis_index("s") * 2 + lax.axis_index("c")
        base = wid * per_w
        for i in range(n_ch):
            off = base + i * _CH
            pltpu.sync_copy(idx_hbm.at[pl.ds(off, _CH)], idx_v)
            pltpu.async_copy(table_hbm.at[idx_v], rows_v, sem).wait()
            pltpu.sync_copy(rows_v, out_hbm.at[pl.ds(off, _CH)])

    return gather_kernel(table, idx)


def _embed(x, xc, emb_cat):
    def body(x_ref, xc_ref, e_ref, o_ref):
        xv = x_ref[...].astype(jnp.int32)
        oh0 = (xv[:, 0:1] == lax.broadcasted_iota(jnp.int32, (_TN, 14), 1))
        oh1 = (xv[:, 1:2] == lax.broadcasted_iota(jnp.int32, (_TN, 5), 1))
        oh2 = (xv[:, 2:3] == lax.broadcasted_iota(jnp.int32, (_TN, 10), 1))
        oh = jnp.concatenate(
            [oh0.astype(jnp.float32), oh1.astype(jnp.float32),
             oh2.astype(jnp.float32)], axis=1)
        e = jnp.dot(oh, e_ref[...], preferred_element_type=jnp.float32)
        o_ref[...] = jnp.concatenate(
            [e, xc_ref[...], jnp.zeros((_TN, 2), jnp.float32)], axis=1)

    return pl.pallas_call(
        body,
        grid=(_N // _TN,),
        in_specs=[
            pl.BlockSpec((_TN, 3), lambda i: (i, 0)),
            pl.BlockSpec((_TN, 34), lambda i: (i, 0)),
            pl.BlockSpec((29, 12), lambda i: (0, 0)),
        ],
        out_specs=pl.BlockSpec((_TN, 48), lambda i: (i, 0)),
        out_shape=jax.ShapeDtypeStruct((_N, 48), jnp.float32),
    )(x, xc, emb_cat)


def _sage_layer(h, neigh, Wg, Ug, bg, Ws, bs, Wn, dp_in, dout, extra=None):
    d_o = 10 if extra is not None else dout

    def body(h_ref, n_ref, Wg_ref, Ug_ref, bg_ref, Ws_ref, bs_ref, Wn_ref,
             *rest):
        o_ref = rest[-1]
        hv = h_ref[...]
        selfp = jnp.dot(hv, Ws_ref[...],
                        preferred_element_type=jnp.float32) + bs_ref[...]
        Wgv = Wg_ref[...]
        Ugv = Ug_ref[...]
        bgv = bg_ref[...]
        hs = jnp.zeros((_TN, 128), jnp.float32)
        c = jnp.zeros((_TN, 128), jnp.float32)
        for t in range(_DEG):
            gates = (jnp.dot(n_ref[t], Wgv, preferred_element_type=jnp.float32)
                     + jnp.dot(hs, Ugv, preferred_element_type=jnp.float32)
                     + bgv)
            gi = jax.nn.sigmoid(gates[:, 0:128])
            gf = jax.nn.sigmoid(gates[:, 128:256])
            gg = jnp.tanh(gates[:, 256:384])
            go = jax.nn.sigmoid(gates[:, 384:512])
            c = gf * c + gi * gg
            hs = go * jnp.tanh(c)
        out = selfp + jnp.dot(hs, Wn_ref[...],
                              preferred_element_type=jnp.float32)
        out = jnp.where(out > 0, out, _NEG * out)
        if extra is not None:
            W1_ref, b1_ref, W2_ref, b2_ref = rest[:4]
            z = jnp.dot(out, W1_ref[...],
                        preferred_element_type=jnp.float32) + b1_ref[...]
            z = jnp.where(z > 0, z, _NEG * z)
            out = jnp.dot(z, W2_ref[...],
                          preferred_element_type=jnp.float32) + b2_ref[...]
        o_ref[...] = out

    in_specs = [
        pl.BlockSpec((_TN, dp_in), lambda i: (i, 0)),
        pl.BlockSpec((_DEG, _TN, dp_in), lambda i: (0, i, 0)),
        pl.BlockSpec((dp_in, 512), lambda i: (0, 0)),
        pl.BlockSpec((128, 512), lambda i: (0, 0)),
        pl.BlockSpec((1, 512), lambda i: (0, 0)),
        pl.BlockSpec((dp_in, dout), lambda i: (0, 0)),
        pl.BlockSpec((1, dout), lambda i: (0, 0)),
        pl.BlockSpec((128, dout), lambda i: (0, 0)),
    ]
    args = [h, neigh, Wg, Ug, bg, Ws, bs, Wn]
    if extra is not None:
        W1, b1, W2, b2 = extra
        in_specs += [
            pl.BlockSpec((dout, 72), lambda i: (0, 0)),
            pl.BlockSpec((1, 72), lambda i: (0, 0)),
            pl.BlockSpec((72, 10), lambda i: (0, 0)),
            pl.BlockSpec((1, 10), lambda i: (0, 0)),
        ]
        args += [W1, b1, W2, b2]

    return pl.pallas_call(
        body,
        grid=(_N // _TN,),
        in_specs=in_specs,
        out_specs=pl.BlockSpec((_TN, d_o), lambda i: (i, 0)),
        out_shape=jax.ShapeDtypeStruct((_N, d_o), jnp.float32),
    )(*args)


def _pack_gates(Wih, Whh, bih, bhh, din, dp_in):
    Wg = jnp.concatenate(
        [jnp.pad(Wih[g * din:(g + 1) * din, :].T,
                 ((0, dp_in - din), (0, 128 - din))) for g in range(4)],
        axis=1)
    Ug = jnp.concatenate(
        [jnp.pad(Whh[g * din:(g + 1) * din, :].T,
                 ((0, 128 - din), (0, 128 - din))) for g in range(4)],
        axis=1)
    b = bih + bhh
    bg = jnp.concatenate(
        [jnp.pad(b[g * din:(g + 1) * din], (0, 128 - din)) for g in range(4)]
    )[None, :]
    return Wg, Ug, bg


def kernel(x, edge_index, params):
    src = edge_index[0].astype(jnp.int32)
    src_tm = src.reshape(_N, _DEG).T.reshape(-1)

    e0, e1, e2 = params['emb0'], params['emb1'], params['emb2']
    emb_cat = jnp.concatenate([
        jnp.pad(e0, ((0, 0), (0, 6))),
        jnp.pad(e1, ((0, 0), (6, 4))),
        jnp.pad(e2, ((0, 0), (8, 0))),
    ], axis=0)

    h = _embed(x[:, :3], x[:, 3:], emb_cat)

    dims = [(46, 48, 64), (64, 64, 80), (80, 80, 96)]
    for l, (din, dp_in, dout) in enumerate(dims):
        Wg, Ug, bg = _pack_gates(
            params['l%d_Wih' % l], params['l%d_Whh' % l],
            params['l%d_bih' % l], params['l%d_bhh' % l], din, dp_in)
        Ws = jnp.pad(params['l%d_Wself' % l].T, ((0, dp_in - din), (0, 0)))
        bs = params['l%d_bself' % l][None, :]
        Wn = jnp.pad(params['l%d_Wneigh' % l].T, ((0, 128 - din), (0, 0)))
        extra = None
        if l == 2:
            extra = (params['lin1_W'].T, params['lin1_b'][None, :],
                     params['lin2_W'].T, params['lin2_b'][None, :])
        neigh = _sc_gather(h, src_tm, dp_in).reshape(_DEG, _N, dp_in)
        h = _sage_layer(h, neigh, Wg, Ug, bg, Ws, bs, Wn, dp_in, dout,
                        extra=extra)
    return h


# speedup vs baseline: 2.6881x; 1.0341x over previous
import functools

import jax
import jax.numpy as jnp
from jax import lax
from jax.experimental import pallas as pl
from jax.experimental.pallas import tpu as pltpu
from jax.experimental.pallas import tpu_sc as plsc

_N = 10000
_DEG = 16
_E = _N * _DEG
_NEG = 0.01

_NW = 32
_CH = 1000
_TN = 1000


def _sc_gather(table, idx, D):
    E = idx.shape[0]
    per_w = E // _NW
    n_ch = per_w // _CH
    mesh = plsc.VectorSubcoreMesh(core_axis_name="c", subcore_axis_name="s")

    @functools.partial(
        pl.kernel,
        out_type=jax.ShapeDtypeStruct((E, D), jnp.float32),
        mesh=mesh,
        scratch_types=[
            pltpu.VMEM((_CH,), jnp.int32),
            pltpu.VMEM((_CH, D), jnp.float32),
            pltpu.SemaphoreType.DMA,
        ],
        compiler_params=pltpu.CompilerParams(use_tc_tiling_on_sc=False),
    )
    def gather_kernel(table_hbm, idx_hbm, out_hbm, idx_v, rows_v, sem):
        wid = lax.axis_index("s") * 2 + lax.axis_index("c")
        base = wid * per_w
        for i in range(n_ch):
            off = base + i * _CH
            pltpu.sync_copy(idx_hbm.at[pl.ds(off, _CH)], idx_v)
            pltpu.async_copy(table_hbm.at[idx_v], rows_v, sem).wait()
            pltpu.sync_copy(rows_v, out_hbm.at[pl.ds(off, _CH)])

    return gather_kernel(table, idx)


def _embed(x, xc, emb_cat):
    def body(x_ref, xc_ref, e_ref, o_ref):
        xv = x_ref[...].astype(jnp.int32)
        oh0 = (xv[:, 0:1] == lax.broadcasted_iota(jnp.int32, (_TN, 14), 1))
        oh1 = (xv[:, 1:2] == lax.broadcasted_iota(jnp.int32, (_TN, 5), 1))
        oh2 = (xv[:, 2:3] == lax.broadcasted_iota(jnp.int32, (_TN, 10), 1))
        oh = jnp.concatenate(
            [oh0.astype(jnp.float32), oh1.astype(jnp.float32),
             oh2.astype(jnp.float32)], axis=1)
        e = jnp.dot(oh, e_ref[...], preferred_element_type=jnp.float32)
        o_ref[...] = jnp.concatenate(
            [e, xc_ref[...], jnp.zeros((_TN, 2), jnp.float32)], axis=1)

    return pl.pallas_call(
        body,
        grid=(_N // _TN,),
        in_specs=[
            pl.BlockSpec((_TN, 3), lambda i: (i, 0)),
            pl.BlockSpec((_TN, 34), lambda i: (i, 0)),
            pl.BlockSpec((29, 12), lambda i: (0, 0)),
        ],
        out_specs=pl.BlockSpec((_TN, 48), lambda i: (i, 0)),
        out_shape=jax.ShapeDtypeStruct((_N, 48), jnp.float32),
    )(x, xc, emb_cat)


def _sage_layer(h, neigh, Wg, Ug, bg, Ws, bs, Wn, dp_in, dg, dout,
                extra=None):
    GP = 4 * dg
    d_o = 10 if extra is not None else dout

    def body(h_ref, n_ref, Wg_ref, Ug_ref, bg_ref, Ws_ref, bs_ref, Wn_ref,
             *rest):
        o_ref = rest[-1]
        hv = h_ref[...]
        selfp = jnp.dot(hv, Ws_ref[...],
                        preferred_element_type=jnp.float32) + bs_ref[...]
        nv = n_ref[...].reshape(_DEG * _TN, dp_in)
        gin = jnp.dot(nv, Wg_ref[...],
                      preferred_element_type=jnp.float32) + bg_ref[...]
        Ugv = Ug_ref[...]
        hs = jnp.zeros((_TN, dg), jnp.float32)
        c = jnp.zeros((_TN, dg), jnp.float32)
        for t in range(_DEG):
            gates = (gin[t * _TN:(t + 1) * _TN, :]
                     + jnp.dot(hs, Ugv, preferred_element_type=jnp.float32))
            s = jax.nn.sigmoid(gates)
            si = s[:, 0:dg]
            sf = s[:, dg:2 * dg]
            sg = s[:, 2 * dg:3 * dg]
            so = s[:, 3 * dg:4 * dg]
            c = sf * c + si * (2.0 * sg - 1.0)
            hs = so * jnp.tanh(c)
        out = selfp + jnp.dot(hs, Wn_ref[...],
                              preferred_element_type=jnp.float32)
        out = jnp.where(out > 0, out, _NEG * out)
        if extra is not None:
            W1_ref, b1_ref, W2_ref, b2_ref = rest[:4]
            z = jnp.dot(out, W1_ref[...],
                        preferred_element_type=jnp.float32) + b1_ref[...]
            z = jnp.where(z > 0, z, _NEG * z)
            out = jnp.dot(z, W2_ref[...],
                          preferred_element_type=jnp.float32) + b2_ref[...]
        o_ref[...] = out

    in_specs = [
        pl.BlockSpec((_TN, dp_in), lambda i: (i, 0)),
        pl.BlockSpec((_DEG, _TN, dp_in), lambda i: (0, i, 0)),
        pl.BlockSpec((dp_in, GP), lambda i: (0, 0)),
        pl.BlockSpec((dg, GP), lambda i: (0, 0)),
        pl.BlockSpec((1, GP), lambda i: (0, 0)),
        pl.BlockSpec((dp_in, dout), lambda i: (0, 0)),
        pl.BlockSpec((1, dout), lambda i: (0, 0)),
        pl.BlockSpec((dg, dout), lambda i: (0, 0)),
    ]
    args = [h, neigh, Wg, Ug, bg, Ws, bs, Wn]
    if extra is not None:
        W1, b1, W2, b2 = extra
        in_specs += [
            pl.BlockSpec((dout, 72), lambda i: (0, 0)),
            pl.BlockSpec((1, 72), lambda i: (0, 0)),
            pl.BlockSpec((72, 10), lambda i: (0, 0)),
            pl.BlockSpec((1, 10), lambda i: (0, 0)),
        ]
        args += [W1, b1, W2, b2]

    return pl.pallas_call(
        body,
        grid=(_N // _TN,),
        in_specs=in_specs,
        out_specs=pl.BlockSpec((_TN, d_o), lambda i: (i, 0)),
        out_shape=jax.ShapeDtypeStruct((_N, d_o), jnp.float32),
    )(*args)


def _pack_gates(Wih, Whh, bih, bhh, din, dp_in, dg):
    sc = jnp.array([1.0, 1.0, 2.0, 1.0], jnp.float32)
    Wg = jnp.concatenate(
        [sc[g] * jnp.pad(Wih[g * din:(g + 1) * din, :].T,
                         ((0, dp_in - din), (0, dg - din))) for g in range(4)],
        axis=1)
    Ug = jnp.concatenate(
        [sc[g] * jnp.pad(Whh[g * din:(g + 1) * din, :].T,
                         ((0, dg - din), (0, dg - din))) for g in range(4)],
        axis=1)
    b = bih + bhh
    bg = jnp.concatenate(
        [sc[g] * jnp.pad(b[g * din:(g + 1) * din], (0, dg - din))
         for g in range(4)])[None, :]
    return Wg, Ug, bg


def kernel(x, edge_index, params):
    src = edge_index[0].astype(jnp.int32)
    src_tm = src.reshape(_N, _DEG).T.reshape(-1)

    e0, e1, e2 = params['emb0'], params['emb1'], params['emb2']
    emb_cat = jnp.concatenate([
        jnp.pad(e0, ((0, 0), (0, 6))),
        jnp.pad(e1, ((0, 0), (6, 4))),
        jnp.pad(e2, ((0, 0), (8, 0))),
    ], axis=0)

    h = _embed(x[:, :3], x[:, 3:], emb_cat)

    dims = [(46, 48, 64, 64), (64, 64, 64, 80), (80, 80, 96, 96)]
    for l, (din, dp_in, dg, dout) in enumerate(dims):
        Wg, Ug, bg = _pack_gates(
            params['l%d_Wih' % l], params['l%d_Whh' % l],
            params['l%d_bih' % l], params['l%d_bhh' % l], din, dp_in, dg)
        Ws = jnp.pad(params['l%d_Wself' % l].T, ((0, dp_in - din), (0, 0)))
        bs = params['l%d_bself' % l][None, :]
        Wn = jnp.pad(params['l%d_Wneigh' % l].T, ((0, dg - din), (0, 0)))
        extra = None
        if l == 2:
            extra = (params['lin1_W'].T, params['lin1_b'][None, :],
                     params['lin2_W'].T, params['lin2_b'][None, :])
        neigh = _sc_gather(h, src_tm, dp_in).reshape(_DEG, _N, dp_in)
        h = _sage_layer(h, neigh, Wg, Ug, bg, Ws, bs, Wn, dp_in, dg, dout,
                        extra=extra)
    return h
